# baseline (speedup 1.0000x reference)
import functools

import jax
import jax.numpy as jnp
from jax import lax
from jax.experimental import pallas as pl
from jax.experimental.pallas import tpu as pltpu
from jax.experimental.pallas import tpu_sc as plsc

D_MODEL = 64
GATHER_W = 128
NUM_CORES = 2
NUM_SUBCORES = 16
NUM_WORKERS = NUM_CORES * NUM_SUBCORES
CHUNK_BLKS = 8
CHUNK = CHUNK_BLKS * GATHER_W


@functools.cache
def _build(n_rows):
    assert n_rows % (NUM_WORKERS * CHUNK) == 0
    rows_per_w = n_rows // NUM_WORKERS
    blks_per_w = rows_per_w // GATHER_W
    n_chunks = rows_per_w // CHUNK

    mesh = plsc.VectorSubcoreMesh(core_axis_name="c", subcore_axis_name="s")

    @functools.partial(
        pl.kernel,
        mesh=mesh,
        compiler_params=pltpu.CompilerParams(use_tc_tiling_on_sc=False),
        out_type=jax.ShapeDtypeStruct((n_rows, D_MODEL), jnp.float32),
        scratch_types=[
            pltpu.VMEM((CHUNK_BLKS, GATHER_W), jnp.int32),
            pltpu.VMEM((CHUNK, D_MODEL), jnp.float32),
            pltpu.SemaphoreType.DMA,
        ],
    )
    def gather_kernel(idx_hbm, table_hbm, out_hbm, idx_v, rows_v, sem):
        wid = lax.axis_index("s") * NUM_CORES + lax.axis_index("c")

        def body(i, carry):
            blk0 = wid * blks_per_w + i * CHUNK_BLKS
            pltpu.sync_copy(idx_hbm.at[pl.ds(blk0, CHUNK_BLKS)], idx_v)
            copies = [
                pltpu.async_copy(
                    table_hbm.at[idx_v.at[j]],
                    rows_v.at[pl.ds(j * GATHER_W, GATHER_W)],
                    sem,
                )
                for j in range(CHUNK_BLKS)
            ]
            for c in copies:
                c.wait()
            row0 = wid * rows_per_w + i * CHUNK
            pltpu.sync_copy(rows_v, out_hbm.at[pl.ds(row0, CHUNK)])
            return carry

        lax.fori_loop(0, n_chunks, body, 0)

    return gather_kernel


def kernel(token_ids, embedding_weight):
    batch, seq = token_ids.shape
    n_rows = batch * seq
    idx = token_ids.reshape(n_rows // GATHER_W, GATHER_W).astype(jnp.int32)
    out = _build(n_rows)(idx, embedding_weight)
    return out.reshape(batch, seq, D_MODEL)


# speedup vs baseline: 1.0168x; 1.0168x over previous
import functools

import jax
import jax.numpy as jnp
from jax import lax
from jax.experimental import pallas as pl
from jax.experimental.pallas import tpu as pltpu
from jax.experimental.pallas import tpu_sc as plsc

D_MODEL = 64
GATHER_W = 128
NUM_CORES = 2
NUM_SUBCORES = 16
NUM_WORKERS = NUM_CORES * NUM_SUBCORES
CHUNK_BLKS = 4
CHUNK = CHUNK_BLKS * GATHER_W


@functools.cache
def _build(n_rows):
    assert n_rows % (NUM_WORKERS * 2 * CHUNK) == 0
    rows_per_w = n_rows // NUM_WORKERS
    blks_per_w = rows_per_w // GATHER_W
    n_chunks = rows_per_w // CHUNK

    mesh = plsc.VectorSubcoreMesh(core_axis_name="c", subcore_axis_name="s")

    @functools.partial(
        pl.kernel,
        mesh=mesh,
        compiler_params=pltpu.CompilerParams(use_tc_tiling_on_sc=False),
        out_type=jax.ShapeDtypeStruct((n_rows, D_MODEL), jnp.float32),
        scratch_types=[
            pltpu.VMEM((2, CHUNK_BLKS, GATHER_W), jnp.int32),
            pltpu.VMEM((2, CHUNK, D_MODEL), jnp.float32),
            pltpu.SemaphoreType.DMA,
            pltpu.SemaphoreType.DMA,
            pltpu.SemaphoreType.DMA,
            pltpu.SemaphoreType.DMA,
            pltpu.SemaphoreType.DMA,
            pltpu.SemaphoreType.DMA,
        ],
    )
    def gather_kernel(idx_hbm, table_hbm, out_hbm, idx_v, rows_v,
                      g0, g1, o0, o1, x0, x1):
        wid = lax.axis_index("s") * NUM_CORES + lax.axis_index("c")
        blk_base = wid * blks_per_w
        row_base = wid * rows_per_w
        gs, os_, xs = (g0, g1), (o0, o1), (x0, x1)

        def fire_idx(i, b):
            pltpu.async_copy(
                idx_hbm.at[pl.ds(blk_base + i * CHUNK_BLKS, CHUNK_BLKS)],
                idx_v.at[b], xs[b])

        def wait_idx(b):
            pltpu.make_async_copy(
                idx_hbm.at[pl.ds(blk_base, CHUNK_BLKS)],
                idx_v.at[b], xs[b]).wait()

        def fire_gathers(b):
            for j in range(CHUNK_BLKS):
                pltpu.async_copy(
                    table_hbm.at[idx_v.at[b, j]],
                    rows_v.at[b, pl.ds(j * GATHER_W, GATHER_W)], gs[b])

        def wait_gathers(b):
            for j in range(CHUNK_BLKS):
                pltpu.make_async_copy(
                    table_hbm.at[idx_v.at[b, j]],
                    rows_v.at[b, pl.ds(j * GATHER_W, GATHER_W)], gs[b]).wait()

        def fire_out(i, b):
            pltpu.async_copy(
                rows_v.at[b],
                out_hbm.at[pl.ds(row_base + i * CHUNK, CHUNK)], os_[b])

        def wait_out(b):
            pltpu.make_async_copy(
                rows_v.at[b],
                out_hbm.at[pl.ds(row_base, CHUNK)], os_[b]).wait()

        fire_idx(0, 0)
        fire_idx(1, 1)
        wait_idx(0)
        fire_gathers(0)
        wait_gathers(0)
        fire_out(0, 0)
        fire_idx(2, 0)
        wait_idx(1)
        fire_gathers(1)

        @pl.loop(2, n_chunks, step=2)
        def _(i0):
            for boff in range(2):
                i = i0 + boff
                b, nb = boff, 1 - boff
                wait_gathers(nb)
                fire_out(i - 1, nb)

                @pl.when(i + 1 < n_chunks)
                def _():
                    fire_idx(i + 1, nb)

                wait_out(b)
                wait_idx(b)
                fire_gathers(b)

        last = (n_chunks - 1) % 2
        wait_gathers(last)
        fire_out(n_chunks - 1, last)
        wait_out(0)
        wait_out(1)

    return gather_kernel


def kernel(token_ids, embedding_weight):
    batch, seq = token_ids.shape
    n_rows = batch * seq
    idx = token_ids.reshape(n_rows // GATHER_W, GATHER_W).astype(jnp.int32)
    out = _build(n_rows)(idx, embedding_weight)
    return out.reshape(batch, seq, D_MODEL)
